# Initial kernel scaffold; baseline (speedup 1.0000x reference)
import jax
import jax.numpy as jnp
from jax.experimental import pallas as pl


def kernel(x, table):
    raise NotImplementedError("write your pallas kernel here")


# speedup vs baseline: 2.7192x; 2.7192x over previous
import functools

import jax
import jax.numpy as jnp
from jax import lax
from jax.experimental import pallas as pl
from jax.experimental.pallas import tpu as pltpu
from jax.experimental.pallas import tpu_sc as plsc

W = 104
L = 16


def _make_kernel(B, S, V, D, n_even, n_odd):
    info = plsc.get_sparse_core_info()
    NC, NS = info.num_cores, info.num_subcores
    NW = NC * NS
    RPT = B // NW

    mesh = plsc.VectorSubcoreMesh(core_axis_name="c", subcore_axis_name="s")

    @functools.partial(
        pl.kernel,
        mesh=mesh,
        out_type=jax.ShapeDtypeStruct((B, n_even, D), jnp.float32),
        compiler_params=pltpu.CompilerParams(use_tc_tiling_on_sc=False),
        scratch_types=[
            pltpu.VMEM((RPT, W), jnp.int32),
            pltpu.VMEM((RPT, W), jnp.int32),
            pltpu.VMEM((W, D), jnp.float32),
            pltpu.VMEM((W, D), jnp.float32),
            pltpu.SemaphoreType.DMA,
            pltpu.SemaphoreType.DMA,
        ],
    )
    def k(xe_h, xo_h, table_h, out_h,
          idxe_v, idxo_v, even_v, odd_v, sem_e, sem_o):
        wid = lax.axis_index("s") * NC + lax.axis_index("c")
        base = wid * RPT
        pltpu.sync_copy(xe_h.at[pl.ds(base, RPT)], idxe_v)
        pltpu.sync_copy(xo_h.at[pl.ds(base, RPT)], idxo_v)

        def row(r, carry):
            cp_e = pltpu.async_copy(table_h.at[idxe_v.at[r]], even_v, sem_e)
            cp_o = pltpu.async_copy(table_h.at[idxo_v.at[r]], odd_v, sem_o)
            cp_e.wait()
            cp_o.wait()

            def add_row(i, c):
                for j in range(D // L):
                    even_v[i, pl.ds(j * L, L)] = (
                        even_v[i, pl.ds(j * L, L)] + odd_v[i, pl.ds(j * L, L)]
                    )
                return c

            lax.fori_loop(0, n_odd, add_row, 0)
            pltpu.sync_copy(even_v.at[pl.ds(0, n_even)], out_h.at[base + r])
            return carry

        lax.fori_loop(0, RPT, row, 0)

    return k


def kernel(x, table):
    B, S = x.shape
    V, D = table.shape
    n_even = (S + 1) // 2
    n_odd = S // 2
    xe = jnp.pad(x[:, ::2], ((0, 0), (0, W - n_even)))
    xo = jnp.pad(x[:, 1::2], ((0, 0), (0, W - n_odd)))
    k = _make_kernel(B, S, V, D, n_even, n_odd)
    return k(xe, xo, table)


# speedup vs baseline: 2.7198x; 1.0002x over previous
import functools

import jax
import jax.numpy as jnp
from jax import lax
from jax.experimental import pallas as pl
from jax.experimental.pallas import tpu as pltpu
from jax.experimental.pallas import tpu_sc as plsc

W = 104
L = 16


def _make_kernel(B, S, V, D, n_even, n_odd):
    info = plsc.get_sparse_core_info()
    NC, NS = info.num_cores, info.num_subcores
    NW = NC * NS
    RPT = B // NW

    mesh = plsc.VectorSubcoreMesh(core_axis_name="c", subcore_axis_name="s")

    @functools.partial(
        pl.kernel,
        mesh=mesh,
        out_type=jax.ShapeDtypeStruct((B, n_even, D), jnp.float32),
        compiler_params=pltpu.CompilerParams(use_tc_tiling_on_sc=False),
        scratch_types=[
            pltpu.VMEM((RPT, W), jnp.int32),
            pltpu.VMEM((RPT, W), jnp.int32),
            pltpu.VMEM((W, D), jnp.float32),
            pltpu.VMEM((W, D), jnp.float32),
            pltpu.VMEM((W, D), jnp.float32),
            pltpu.VMEM((W, D), jnp.float32),
            pltpu.SemaphoreType.DMA,
            pltpu.SemaphoreType.DMA,
            pltpu.SemaphoreType.DMA,
            pltpu.SemaphoreType.DMA,
            pltpu.SemaphoreType.DMA,
            pltpu.SemaphoreType.DMA,
        ],
    )
    def k(xe_h, xo_h, table_h, out_h,
          idxe_v, idxo_v, ev0, ev1, ov0, ov1,
          ge0, ge1, go0, go1, so0, so1):
        evs, ovs = (ev0, ev1), (ov0, ov1)
        ges, gos, sos = (ge0, ge1), (go0, go1), (so0, so1)
        wid = lax.axis_index("s") * NC + lax.axis_index("c")
        base = wid * RPT
        pltpu.sync_copy(xe_h.at[pl.ds(base, RPT)], idxe_v)
        pltpu.sync_copy(xo_h.at[pl.ds(base, RPT)], idxo_v)

        def issue(row, b):
            pltpu.async_copy(table_h.at[idxe_v.at[row - base]], evs[b], ges[b])
            pltpu.async_copy(table_h.at[idxo_v.at[row - base]], ovs[b], gos[b])

        def wait_gather(b):
            dummy = table_h.at[pl.ds(0, W)]
            pltpu.make_async_copy(dummy, evs[b], ges[b]).wait()
            pltpu.make_async_copy(dummy, ovs[b], gos[b]).wait()

        def merge(b):
            ev, ov = evs[b], ovs[b]

            def f(t, c):
                for u in range(3):
                    i = 3 * t + u
                    for j in range(D // L):
                        plsc.addupdate(
                            ev.at[i, pl.ds(j * L, L)],
                            ov[i, pl.ds(j * L, L)],
                        )
                return c

            lax.fori_loop(0, n_odd // 3, f, 0)

        def start_out(row, b):
            pltpu.async_copy(
                evs[b].at[pl.ds(0, n_even)], out_h.at[row], sos[b])

        def wait_out(b):
            pltpu.make_async_copy(
                evs[b].at[pl.ds(0, n_even)], out_h.at[base], sos[b]).wait()

        issue(base, 0)

        def pair(kk, c):
            r0 = base + 2 * kk

            @pl.when(kk > 0)
            def _():
                wait_out(1)

            issue(r0 + 1, 1)
            wait_gather(0)
            merge(0)
            start_out(r0, 0)

            @pl.when(kk < RPT // 2 - 1)
            def _():
                wait_out(0)
                issue(r0 + 2, 0)

            wait_gather(1)
            merge(1)
            start_out(r0 + 1, 1)
            return c

        lax.fori_loop(0, RPT // 2, pair, 0)
        wait_out(0)
        wait_out(1)

    return k


def kernel(x, table):
    B, S = x.shape
    V, D = table.shape
    n_even = (S + 1) // 2
    n_odd = S // 2
    xe = jnp.pad(x[:, ::2], ((0, 0), (0, W - n_even)))
    xo = jnp.pad(x[:, 1::2], ((0, 0), (0, W - n_odd)))
    k = _make_kernel(B, S, V, D, n_even, n_odd)
    return k(xe, xo, table)


# speedup vs baseline: 2.8980x; 1.0655x over previous
import functools

import jax
import jax.numpy as jnp
from jax import lax
from jax.experimental import pallas as pl
from jax.experimental.pallas import tpu as pltpu
from jax.experimental.pallas import tpu_sc as plsc

W = 104
L = 16


def _make_kernel(B, S, V, D, n_even, n_odd):
    info = plsc.get_sparse_core_info()
    NC, NS = info.num_cores, info.num_subcores
    NW = NC * NS
    RPT = B // NW

    mesh = plsc.VectorSubcoreMesh(core_axis_name="c", subcore_axis_name="s")

    @functools.partial(
        pl.kernel,
        mesh=mesh,
        out_type=jax.ShapeDtypeStruct((B, n_even, D), jnp.float32),
        compiler_params=pltpu.CompilerParams(use_tc_tiling_on_sc=False),
        scratch_types=[
            pltpu.VMEM((RPT, W), jnp.int32),
            pltpu.VMEM((RPT, W), jnp.int32),
            pltpu.VMEM((W, D), jnp.float32),
            pltpu.VMEM((W, D), jnp.float32),
            pltpu.VMEM((W, D), jnp.float32),
            pltpu.VMEM((W, D), jnp.float32),
            pltpu.SemaphoreType.DMA,
            pltpu.SemaphoreType.DMA,
            pltpu.SemaphoreType.DMA,
            pltpu.SemaphoreType.DMA,
            pltpu.SemaphoreType.DMA,
            pltpu.SemaphoreType.DMA,
        ],
    )
    def k(xe_h, xo_h, table_h, out_h,
          idxe_v, idxo_v, ev0, ev1, ov0, ov1,
          ge0, ge1, go0, go1, so0, so1):
        evs, ovs = (ev0, ev1), (ov0, ov1)
        ges, gos, sos = (ge0, ge1), (go0, go1), (so0, so1)
        wid = lax.axis_index("s") * NC + lax.axis_index("c")
        base = wid * RPT
        pltpu.sync_copy(xe_h.at[pl.ds(base, RPT)], idxe_v)
        pltpu.sync_copy(xo_h.at[pl.ds(base, RPT)], idxo_v)

        def issue(row, b):
            pltpu.async_copy(table_h.at[idxe_v.at[row - base]], evs[b], ges[b])
            pltpu.async_copy(table_h.at[idxo_v.at[row - base]], ovs[b], gos[b])

        def wait_gather(b):
            dummy = table_h.at[pl.ds(0, W)]
            pltpu.make_async_copy(dummy, evs[b], ges[b]).wait()
            pltpu.make_async_copy(dummy, ovs[b], gos[b]).wait()

        def merge(b):
            ev, ov = evs[b], ovs[b]

            def f(t, c):
                for u in range(3):
                    i = 3 * t + u
                    for j in range(D // L):
                        plsc.addupdate(
                            ev.at[i, pl.ds(j * L, L)],
                            ov[i, pl.ds(j * L, L)],
                        )
                return c

            lax.fori_loop(0, n_odd // 3, f, 0)

        def start_out(row, b):
            pltpu.async_copy(
                evs[b].at[pl.ds(0, n_even)], out_h.at[row], sos[b])

        def wait_out(b):
            pltpu.make_async_copy(
                evs[b].at[pl.ds(0, n_even)], out_h.at[base], sos[b]).wait()

        issue(base, 0)

        def pair(kk, c):
            r0 = base + 2 * kk

            @pl.when(kk > 0)
            def _():
                wait_out(1)

            issue(r0 + 1, 1)
            wait_gather(0)
            merge(0)
            start_out(r0, 0)

            @pl.when(kk < RPT // 2 - 1)
            def _():
                wait_out(0)
                issue(r0 + 2, 0)

            wait_gather(1)
            merge(1)
            start_out(r0 + 1, 1)
            return c

        lax.fori_loop(0, RPT // 2, pair, 0)
        wait_out(0)
        wait_out(1)

    return k


def _relayout(table):
    D, V = table.shape
    BC = 2048
    G = (V + BC - 1) // BC

    def tr(x_ref, o_ref):
        y = x_ref[...].T.reshape(BC // 2, 2, D)
        z = jnp.concatenate([y[:, 0, :], y[:, 1, :]], axis=1)
        o_ref[...] = z.reshape(-1)

    flat = pl.pallas_call(
        tr,
        grid=(G,),
        in_specs=[pl.BlockSpec((D, BC), lambda i: (0, i))],
        out_specs=pl.BlockSpec((BC * D,), lambda i: (i,)),
        out_shape=jax.ShapeDtypeStruct((V * D,), jnp.float32),
    )(table)
    return flat


def kernel(x, table):
    B, S = x.shape
    V, D = table.shape
    n_even = (S + 1) // 2
    n_odd = S // 2
    table = _relayout(table.T).reshape(V, D)
    xe = jnp.pad(x[:, ::2], ((0, 0), (0, W - n_even)))
    xo = jnp.pad(x[:, 1::2], ((0, 0), (0, W - n_odd)))
    k = _make_kernel(B, S, V, D, n_even, n_odd)
    return k(xe, xo, table)


# speedup vs baseline: 3.3435x; 1.1537x over previous
import functools

import jax
import jax.numpy as jnp
from jax import lax
from jax.experimental import pallas as pl
from jax.experimental.pallas import tpu as pltpu
from jax.experimental.pallas import tpu_sc as plsc

W = 104
L = 16


def _make_kernel(B, S, V, D, n_even, n_odd):
    info = plsc.get_sparse_core_info()
    NC, NS = info.num_cores, info.num_subcores
    NW = NC * NS
    RPT = B // NW

    mesh = plsc.VectorSubcoreMesh(core_axis_name="c", subcore_axis_name="s")

    @functools.partial(
        pl.kernel,
        mesh=mesh,
        out_type=jax.ShapeDtypeStruct((B, n_even, D), jnp.float32),
        compiler_params=pltpu.CompilerParams(use_tc_tiling_on_sc=False),
        scratch_types=[
            pltpu.VMEM((RPT, W), jnp.int32),
            pltpu.VMEM((RPT, W), jnp.int32),
            pltpu.VMEM((W, D), jnp.float32),
            pltpu.VMEM((W, D), jnp.float32),
            pltpu.VMEM((W, D), jnp.float32),
            pltpu.VMEM((W, D), jnp.float32),
            pltpu.SemaphoreType.DMA,
            pltpu.SemaphoreType.DMA,
            pltpu.SemaphoreType.DMA,
            pltpu.SemaphoreType.DMA,
            pltpu.SemaphoreType.DMA,
            pltpu.SemaphoreType.DMA,
        ],
    )
    def k(xe_h, xo_h, table_h, out_h,
          idxe_v, idxo_v, ev0, ev1, ov0, ov1,
          ge0, ge1, go0, go1, so0, so1):
        evs, ovs = (ev0, ev1), (ov0, ov1)
        ges, gos, sos = (ge0, ge1), (go0, go1), (so0, so1)
        wid = lax.axis_index("s") * NC + lax.axis_index("c")
        base = wid * RPT
        pltpu.sync_copy(xe_h.at[pl.ds(base, RPT)], idxe_v)
        pltpu.sync_copy(xo_h.at[pl.ds(base, RPT)], idxo_v)

        def issue(row, b):
            pltpu.async_copy(table_h.at[idxe_v.at[row - base]], evs[b], ges[b])
            pltpu.async_copy(table_h.at[idxo_v.at[row - base]], ovs[b], gos[b])

        def wait_gather(b):
            dummy = table_h.at[pl.ds(0, W)]
            pltpu.make_async_copy(dummy, evs[b], ges[b]).wait()
            pltpu.make_async_copy(dummy, ovs[b], gos[b]).wait()

        def merge(b):
            ev, ov = evs[b], ovs[b]

            def f(t, c):
                for u in range(3):
                    i = 3 * t + u
                    for j in range(D // L):
                        plsc.addupdate(
                            ev.at[i, pl.ds(j * L, L)],
                            ov[i, pl.ds(j * L, L)],
                        )
                return c

            lax.fori_loop(0, n_odd // 3, f, 0)

        def start_out(row, b):
            pltpu.async_copy(
                evs[b].at[pl.ds(0, n_even)], out_h.at[row], sos[b])

        def wait_out(b):
            pltpu.make_async_copy(
                evs[b].at[pl.ds(0, n_even)], out_h.at[base], sos[b]).wait()

        issue(base, 0)

        def pair(kk, c):
            r0 = base + 2 * kk

            @pl.when(kk > 0)
            def _():
                wait_out(1)

            issue(r0 + 1, 1)
            wait_gather(0)
            merge(0)
            start_out(r0, 0)

            @pl.when(kk < RPT // 2 - 1)
            def _():
                wait_out(0)
                issue(r0 + 2, 0)

            wait_gather(1)
            merge(1)
            start_out(r0 + 1, 1)
            return c

        lax.fori_loop(0, RPT // 2, pair, 0)
        wait_out(0)
        wait_out(1)

    return k


def _relayout(table):
    D, V = table.shape
    BC = 4096
    G = (V + BC - 1) // BC

    def tr(x_ref, o_ref):
        y = x_ref[...].T.reshape(BC // 2, 2, D)
        z = jnp.concatenate([y[:, 0, :], y[:, 1, :]], axis=1)
        o_ref[...] = z.reshape(-1)

    flat = pl.pallas_call(
        tr,
        grid=(G,),
        in_specs=[pl.BlockSpec((D, BC), lambda i: (0, i))],
        out_specs=pl.BlockSpec((BC * D,), lambda i: (i,)),
        out_shape=jax.ShapeDtypeStruct((V * D,), jnp.float32),
    )(table)
    return flat


def kernel(x, table):
    B, S = x.shape
    V, D = table.shape
    n_even = (S + 1) // 2
    n_odd = S // 2
    table = _relayout(table.T).reshape(V, D)
    xe = jnp.pad(x[:, ::2], ((0, 0), (0, W - n_even)))
    xo = jnp.pad(x[:, 1::2], ((0, 0), (0, W - n_odd)))
    k = _make_kernel(B, S, V, D, n_even, n_odd)
    return k(xe, xo, table)


# speedup vs baseline: 3.4416x; 1.0293x over previous
import functools

import jax
import jax.numpy as jnp
from jax import lax
from jax.experimental import pallas as pl
from jax.experimental.pallas import tpu as pltpu
from jax.experimental.pallas import tpu_sc as plsc

W = 104
L = 16


def _make_kernel(B, S, V, D, n_even, n_odd):
    info = plsc.get_sparse_core_info()
    NC, NS = info.num_cores, info.num_subcores
    NW = NC * NS
    RPT = B // NW

    mesh = plsc.VectorSubcoreMesh(core_axis_name="c", subcore_axis_name="s")

    @functools.partial(
        pl.kernel,
        mesh=mesh,
        out_type=jax.ShapeDtypeStruct((B, n_even, D), jnp.float32),
        compiler_params=pltpu.CompilerParams(use_tc_tiling_on_sc=False),
        scratch_types=[
            pltpu.VMEM((RPT, W), jnp.int32),
            pltpu.VMEM((RPT, W), jnp.int32),
            pltpu.VMEM((W, D), jnp.float32),
            pltpu.VMEM((W, D), jnp.float32),
            pltpu.VMEM((W, D), jnp.float32),
            pltpu.VMEM((W, D), jnp.float32),
            pltpu.SemaphoreType.DMA,
            pltpu.SemaphoreType.DMA,
            pltpu.SemaphoreType.DMA,
            pltpu.SemaphoreType.DMA,
            pltpu.SemaphoreType.DMA,
            pltpu.SemaphoreType.DMA,
        ],
    )
    def k(xe_h, xo_h, table_h, out_h,
          idxe_v, idxo_v, ev0, ev1, ov0, ov1,
          ge0, ge1, go0, go1, so0, so1):
        evs, ovs = (ev0, ev1), (ov0, ov1)
        ges, gos, sos = (ge0, ge1), (go0, go1), (so0, so1)
        wid = lax.axis_index("s") * NC + lax.axis_index("c")
        base = wid * RPT
        pltpu.sync_copy(xe_h.at[pl.ds(base, RPT)], idxe_v)
        pltpu.sync_copy(xo_h.at[pl.ds(base, RPT)], idxo_v)

        def issue(row, b):
            pltpu.async_copy(table_h.at[idxe_v.at[row - base]], evs[b], ges[b])
            pltpu.async_copy(table_h.at[idxo_v.at[row - base]], ovs[b], gos[b])

        def wait_gather(b):
            dummy = table_h.at[pl.ds(0, W)]
            pltpu.make_async_copy(dummy, evs[b], ges[b]).wait()
            pltpu.make_async_copy(dummy, ovs[b], gos[b]).wait()

        def merge(b):
            ev, ov = evs[b], ovs[b]

            def f(t, c):
                for u in range(3):
                    i = 3 * t + u
                    for j in range(D // L):
                        plsc.addupdate(
                            ev.at[i, pl.ds(j * L, L)],
                            ov[i, pl.ds(j * L, L)],
                        )
                return c

            lax.fori_loop(0, n_odd // 3, f, 0)

        def start_out(row, b):
            pltpu.async_copy(
                evs[b].at[pl.ds(0, n_even)], out_h.at[row], sos[b])

        def wait_out(b):
            pltpu.make_async_copy(
                evs[b].at[pl.ds(0, n_even)], out_h.at[base], sos[b]).wait()

        issue(base, 0)

        def pair(kk, c):
            r0 = base + 2 * kk

            @pl.when(kk > 0)
            def _():
                wait_out(1)

            issue(r0 + 1, 1)
            wait_gather(0)
            merge(0)
            start_out(r0, 0)

            @pl.when(kk < RPT // 2 - 1)
            def _():
                wait_out(0)
                issue(r0 + 2, 0)

            wait_gather(1)
            merge(1)
            start_out(r0 + 1, 1)
            return c

        lax.fori_loop(0, RPT // 2, pair, 0)
        wait_out(0)
        wait_out(1)

    return k


def _relayout(table):
    D, V = table.shape
    BC = 8192
    G = (V + BC - 1) // BC

    def tr(x_ref, o_ref):
        y = x_ref[...].T.reshape(BC // 2, 2, D)
        z = jnp.concatenate([y[:, 0, :], y[:, 1, :]], axis=1)
        o_ref[...] = z.reshape(-1)

    flat = pl.pallas_call(
        tr,
        grid=(G,),
        in_specs=[pl.BlockSpec((D, BC), lambda i: (0, i))],
        out_specs=pl.BlockSpec((BC * D,), lambda i: (i,)),
        out_shape=jax.ShapeDtypeStruct((V * D,), jnp.float32),
    )(table)
    return flat


def kernel(x, table):
    B, S = x.shape
    V, D = table.shape
    n_even = (S + 1) // 2
    n_odd = S // 2
    table = _relayout(table.T).reshape(V, D)
    xe = jnp.pad(x[:, ::2], ((0, 0), (0, W - n_even)))
    xo = jnp.pad(x[:, 1::2], ((0, 0), (0, W - n_odd)))
    k = _make_kernel(B, S, V, D, n_even, n_odd)
    return k(xe, xo, table)


# speedup vs baseline: 3.4569x; 1.0044x over previous
import functools

import jax
import jax.numpy as jnp
from jax import lax
from jax.experimental import pallas as pl
from jax.experimental.pallas import tpu as pltpu
from jax.experimental.pallas import tpu_sc as plsc

W = 104
L = 16


def _make_kernel(B, S, V, D, n_even, n_odd):
    info = plsc.get_sparse_core_info()
    NC, NS = info.num_cores, info.num_subcores
    NW = NC * NS
    RPT = B // NW

    mesh = plsc.VectorSubcoreMesh(core_axis_name="c", subcore_axis_name="s")

    @functools.partial(
        pl.kernel,
        mesh=mesh,
        out_type=jax.ShapeDtypeStruct((B, n_even, D), jnp.float32),
        compiler_params=pltpu.CompilerParams(use_tc_tiling_on_sc=False),
        scratch_types=[
            pltpu.VMEM((RPT, W), jnp.int32),
            pltpu.VMEM((RPT, W), jnp.int32),
            pltpu.VMEM((W, D), jnp.float32),
            pltpu.VMEM((W, D), jnp.float32),
            pltpu.VMEM((W, D), jnp.float32),
            pltpu.VMEM((W, D), jnp.float32),
            pltpu.SemaphoreType.DMA,
            pltpu.SemaphoreType.DMA,
            pltpu.SemaphoreType.DMA,
            pltpu.SemaphoreType.DMA,
            pltpu.SemaphoreType.DMA,
            pltpu.SemaphoreType.DMA,
        ],
    )
    def k(xe_h, xo_h, table_h, out_h,
          idxe_v, idxo_v, ev0, ev1, ov0, ov1,
          ge0, ge1, go0, go1, so0, so1):
        evs, ovs = (ev0, ev1), (ov0, ov1)
        ges, gos, sos = (ge0, ge1), (go0, go1), (so0, so1)
        wid = lax.axis_index("s") * NC + lax.axis_index("c")
        base = wid * RPT
        pltpu.sync_copy(xe_h.at[pl.ds(base, RPT)], idxe_v)
        pltpu.sync_copy(xo_h.at[pl.ds(base, RPT)], idxo_v)

        def issue(row, b):
            pltpu.async_copy(table_h.at[idxe_v.at[row - base]], evs[b], ges[b])
            pltpu.async_copy(table_h.at[idxo_v.at[row - base]], ovs[b], gos[b])

        def wait_gather(b):
            dummy = table_h.at[pl.ds(0, W)]
            pltpu.make_async_copy(dummy, evs[b], ges[b]).wait()
            pltpu.make_async_copy(dummy, ovs[b], gos[b]).wait()

        def merge(b):
            ev, ov = evs[b], ovs[b]

            def f(t, c):
                for u in range(3):
                    i = 3 * t + u
                    for j in range(D // L):
                        plsc.addupdate(
                            ev.at[i, pl.ds(j * L, L)],
                            ov[i, pl.ds(j * L, L)],
                        )
                return c

            lax.fori_loop(0, n_odd // 3, f, 0)

        def start_out(row, b):
            pltpu.async_copy(
                evs[b].at[pl.ds(0, n_even)], out_h.at[row], sos[b])

        def wait_out(b):
            pltpu.make_async_copy(
                evs[b].at[pl.ds(0, n_even)], out_h.at[base], sos[b]).wait()

        issue(base, 0)

        def pair(kk, c):
            r0 = base + 2 * kk

            @pl.when(kk > 0)
            def _():
                wait_out(1)

            issue(r0 + 1, 1)
            wait_gather(0)
            merge(0)
            start_out(r0, 0)

            @pl.when(kk < RPT // 2 - 1)
            def _():
                wait_out(0)
                issue(r0 + 2, 0)

            wait_gather(1)
            merge(1)
            start_out(r0 + 1, 1)
            return c

        lax.fori_loop(0, RPT // 2, pair, 0)
        wait_out(0)
        wait_out(1)

    return k


def _relayout(table):
    D, V = table.shape
    BC = 16384
    G = (V + BC - 1) // BC

    def tr(x_ref, o_ref):
        y = x_ref[...].T.reshape(BC // 2, 2, D)
        z = jnp.concatenate([y[:, 0, :], y[:, 1, :]], axis=1)
        o_ref[...] = z.reshape(-1)

    flat = pl.pallas_call(
        tr,
        grid=(G,),
        in_specs=[pl.BlockSpec((D, BC), lambda i: (0, i))],
        out_specs=pl.BlockSpec((BC * D,), lambda i: (i,)),
        out_shape=jax.ShapeDtypeStruct((V * D,), jnp.float32),
    )(table)
    return flat


def kernel(x, table):
    B, S = x.shape
    V, D = table.shape
    n_even = (S + 1) // 2
    n_odd = S // 2
    table = _relayout(table.T).reshape(V, D)
    xe = jnp.pad(x[:, ::2], ((0, 0), (0, W - n_even)))
    xo = jnp.pad(x[:, 1::2], ((0, 0), (0, W - n_odd)))
    k = _make_kernel(B, S, V, D, n_even, n_odd)
    return k(xe, xo, table)
